# Initial kernel scaffold; baseline (speedup 1.0000x reference)
import jax
import jax.numpy as jnp
from jax.experimental import pallas as pl


def kernel(coords, values):
    raise NotImplementedError("write your pallas kernel here")


# speedup vs baseline: 30.2875x; 30.2875x over previous
import functools

import jax
import jax.numpy as jnp
from jax import lax
from jax.experimental import pallas as pl
from jax.experimental.pallas import tpu as pltpu
from jax.experimental.pallas import tpu_sc as plsc


_FLUSH = 128


def _fps_body(npoint, x_ref, y_ref, z_ref, idx_ref, cx_ref, cy_ref, cz_ref):
    B, N = x_ref.shape
    x = x_ref[...]
    y = y_ref[...]
    z = z_ref[...]
    lane = lax.broadcasted_iota(jnp.int32, (B, N), 1)
    lane_f = lax.broadcasted_iota(jnp.int32, (B, _FLUSH), 1)
    boffs = lax.broadcasted_iota(jnp.int32, (B, 1), 0) * N

    dist0 = jnp.full((B, N), 1e10, dtype=jnp.float32)
    far0 = jnp.zeros((B, 1), dtype=jnp.int32)
    cx0 = x[:, 0:1]
    cy0 = y[:, 0:1]
    cz0 = z[:, 0:1]

    bi0 = jnp.zeros((B, _FLUSH), dtype=jnp.int32)
    bf0 = jnp.zeros((B, _FLUSH), dtype=jnp.float32)

    def body(j, carry):
        dist, far, cx, cy, cz, bi, bx, by, bz = carry
        sel = lane_f == j
        bi = jnp.where(sel, far + boffs, bi)
        bx = jnp.where(sel, cx, bx)
        by = jnp.where(sel, cy, by)
        bz = jnp.where(sel, cz, bz)
        dx = x - cx
        dy = y - cy
        dz = z - cz
        d = (dx * dx + dz * dz) + dy * dy
        dist = jnp.minimum(dist, d)
        m = jnp.max(dist, axis=1, keepdims=True)
        nf = jnp.min(jnp.where(dist == m, lane, N), axis=1, keepdims=True)
        oh = lane == nf
        zero = jnp.zeros_like(x)
        ncx = jnp.sum(jnp.where(oh, x, zero), axis=1, keepdims=True)
        ncy = jnp.sum(jnp.where(oh, y, zero), axis=1, keepdims=True)
        ncz = jnp.sum(jnp.where(oh, z, zero), axis=1, keepdims=True)
        return (dist, nf, ncx, ncy, ncz, bi, bx, by, bz)

    carry = (dist0, far0, cx0, cy0, cz0)
    for blk in range(npoint // _FLUSH):
        carry = lax.fori_loop(0, _FLUSH, body, carry + (bi0, bf0, bf0, bf0))
        dist, far, cx, cy, cz, bi, bx, by, bz = carry
        sl = slice(blk * _FLUSH, (blk + 1) * _FLUSH)
        idx_ref[:, sl] = bi
        cx_ref[:, sl] = bx
        cy_ref[:, sl] = by
        cz_ref[:, sl] = bz
        carry = (dist, far, cx, cy, cz)


def _run_fps(x, y, z, npoint):
    B, N = x.shape
    out_shape = [jax.ShapeDtypeStruct((B, npoint), jnp.int32)] + [
        jax.ShapeDtypeStruct((B, npoint), jnp.float32)
    ] * 3
    return pl.pallas_call(
        functools.partial(_fps_body, npoint),
        out_shape=out_shape,
    )(x, y, z)


def _make_gather(n_rows, D, n_idx, chunk=128):
    NW = 32
    per_w = n_idx // (NW * chunk)
    mesh = plsc.VectorSubcoreMesh(core_axis_name="c", subcore_axis_name="s")

    @functools.partial(
        pl.kernel,
        mesh=mesh,
        compiler_params=pltpu.CompilerParams(use_tc_tiling_on_sc=False),
        out_type=jax.ShapeDtypeStruct((NW, per_w, chunk, D), jnp.float32),
        scratch_types=[
            pltpu.VMEM((per_w, chunk), jnp.int32),
            pltpu.VMEM((per_w, chunk, D), jnp.float32),
            pltpu.SemaphoreType.DMA,
        ],
    )
    def g(vals_hbm, idx_hbm, out_hbm, idx_v, rows_v, sem):
        wid = lax.axis_index("s") * 2 + lax.axis_index("c")
        pltpu.sync_copy(idx_hbm.at[wid], idx_v)
        cps = [
            pltpu.async_copy(vals_hbm.at[idx_v.at[j]], rows_v.at[j], sem)
            for j in range(per_w)
        ]
        for cp in cps:
            cp.wait()
        pltpu.sync_copy(rows_v, out_hbm.at[wid])

    return g


def kernel(coords, values):
    B, C, N = coords.shape
    _, D, _ = values.shape
    npoint = N // 2

    x = coords[:, 0, :]
    y = coords[:, 1, :]
    z = coords[:, 2, :]
    idx_g, cx, cy, cz = _run_fps(x, y, z, npoint)
    new_coords = jnp.stack([cx, cy, cz], axis=1)

    vals_rows = jnp.transpose(values, (0, 2, 1)).reshape(B * N, D)
    chunk = 128
    idx3 = idx_g.reshape(32, (B * npoint) // (32 * chunk), chunk)
    gathered = _make_gather(B * N, D, B * npoint, chunk)(vals_rows, idx3)
    new_values = jnp.transpose(gathered.reshape(B, npoint, D), (0, 2, 1))
    return new_coords, new_values


# speedup vs baseline: 49.2690x; 1.6267x over previous
import functools

import jax
import jax.numpy as jnp
from jax import lax
from jax.experimental import pallas as pl
from jax.experimental.pallas import tpu as pltpu
from jax.experimental.pallas import tpu_sc as plsc


_FLUSH = 128
_TAIL = 128


_W = 128
_GROUPS = 4


def _fold(a, b):
    keep = a[0] >= b[0]
    return tuple(jnp.where(keep, ai, bi) for ai, bi in zip(a, b))


def _fold_tb(a, b):
    keep = (a[0] > b[0]) | ((a[0] == b[0]) & (a[1] < b[1]))
    return tuple(jnp.where(keep, ai, bi) for ai, bi in zip(a, b))


def _fps_body(npoint, x_ref, y_ref, z_ref, idx_ref, cx_ref, cy_ref, cz_ref, dist_ref):
    B, N = x_ref.shape
    ncols = N // _W
    cpg = ncols // _GROUPS
    lane_f = lax.broadcasted_iota(jnp.int32, (B, _FLUSH), 1)
    boffs = lax.broadcasted_iota(jnp.int32, (B, 1), 0) * N
    boffs_f = boffs.astype(jnp.float32)
    lane_wf = lax.broadcasted_iota(jnp.int32, (B, _W), 1).astype(jnp.float32) + boffs_f

    dist_ref[...] = jnp.full((B, N), 1e10, dtype=jnp.float32)
    far0 = jnp.zeros((B, 1), dtype=jnp.float32) + boffs_f
    cx0 = x_ref[:, 0:1]
    cy0 = y_ref[:, 0:1]
    cz0 = z_ref[:, 0:1]

    bf0 = jnp.zeros((B, _FLUSH), dtype=jnp.float32)
    bi0 = bf0

    def body(j, carry):
        far, cx, cy, cz, bi, bx, by, bz = carry
        sel = lane_f == j
        bi = jnp.where(sel, far, bi)
        bx = jnp.where(sel, cx, bx)
        by = jnp.where(sel, cy, by)
        bz = jnp.where(sel, cz, bz)
        bests = []
        for g in range(_GROUPS):
            best = None
            for c in range(g * cpg, (g + 1) * cpg):
                sl = slice(c * _W, (c + 1) * _W)
                xc = x_ref[:, sl]
                yc = y_ref[:, sl]
                zc = z_ref[:, sl]
                dx = xc - cx
                dy = yc - cy
                dz = zc - cz
                d = (dx * dx + dz * dz) + dy * dy
                dc = jnp.minimum(dist_ref[:, sl], d)
                dist_ref[:, sl] = dc
                col = (dc, jnp.full((B, _W), float(c), jnp.float32), xc, yc, zc)
                best = col if best is None else _fold(best, col)
            bests.append(best)
        best = _fold(_fold(bests[0], bests[1]), _fold(bests[2], bests[3]))
        v, ci, px, py, pz = best
        gi = ci * float(_W) + lane_wf
        m = jnp.max(v, axis=1, keepdims=True)
        eq = v == m
        nf = jnp.min(jnp.where(eq, gi, jnp.float32(2.0**30)), axis=1, keepdims=True)
        oh = gi == nf
        zero = jnp.zeros_like(v)
        ncx = jnp.sum(jnp.where(oh, px, zero), axis=1, keepdims=True)
        ncy = jnp.sum(jnp.where(oh, py, zero), axis=1, keepdims=True)
        ncz = jnp.sum(jnp.where(oh, pz, zero), axis=1, keepdims=True)
        return (nf, ncx, ncy, ncz, bi, bx, by, bz)

    def body4(jj, carry):
        for u in range(4):
            carry = body(jj * 4 + u, carry)
        return carry

    carry = (far0, cx0, cy0, cz0)
    for blk in range(npoint // _FLUSH):
        carry = lax.fori_loop(0, _FLUSH // 4, body4, carry + (bi0, bf0, bf0, bf0))
        far, cx, cy, cz, bi, bx, by, bz = carry
        sl = slice(blk * _FLUSH, (blk + 1) * _FLUSH)
        idx_ref[:, sl] = bi.astype(jnp.int32)
        cx_ref[:, sl] = bx
        cy_ref[:, sl] = by
        cz_ref[:, sl] = bz
        carry = (far, cx, cy, cz)


def _run_fps(x, y, z, npoint):
    B, N = x.shape
    out_shape = [jax.ShapeDtypeStruct((B, npoint), jnp.int32)] + [
        jax.ShapeDtypeStruct((B, npoint), jnp.float32)
    ] * 3
    return pl.pallas_call(
        functools.partial(_fps_body, npoint),
        out_shape=out_shape,
        scratch_shapes=[pltpu.VMEM((B, N), jnp.float32)],
    )(x, y, z)


def _make_gather(n_rows, D, n_idx, chunk=128):
    NW = 32
    per_w = n_idx // (NW * chunk)
    mesh = plsc.VectorSubcoreMesh(core_axis_name="c", subcore_axis_name="s")

    @functools.partial(
        pl.kernel,
        mesh=mesh,
        compiler_params=pltpu.CompilerParams(use_tc_tiling_on_sc=False),
        out_type=jax.ShapeDtypeStruct((NW, per_w, chunk, D), jnp.float32),
        scratch_types=[
            pltpu.VMEM((per_w, chunk), jnp.int32),
            pltpu.VMEM((per_w, chunk, D), jnp.float32),
            pltpu.SemaphoreType.DMA,
        ],
    )
    def g(vals_hbm, idx_hbm, out_hbm, idx_v, rows_v, sem):
        wid = lax.axis_index("s") * 2 + lax.axis_index("c")
        pltpu.sync_copy(idx_hbm.at[wid], idx_v)
        cps = [
            pltpu.async_copy(vals_hbm.at[idx_v.at[j]], rows_v.at[j], sem)
            for j in range(per_w)
        ]
        for cp in cps:
            cp.wait()
        pltpu.sync_copy(rows_v, out_hbm.at[wid])

    return g


def kernel(coords, values):
    B, C, N = coords.shape
    _, D, _ = values.shape
    npoint = N // 2

    x = coords[:, 0, :]
    y = coords[:, 1, :]
    z = coords[:, 2, :]
    idx_g, cx, cy, cz = _run_fps(x, y, z, npoint)
    new_coords = jnp.stack([cx, cy, cz], axis=1)

    vals_rows = jnp.transpose(values, (0, 2, 1)).reshape(B * N, D)
    chunk = 128
    idx3 = idx_g.reshape(32, (B * npoint) // (32 * chunk), chunk)
    gathered = _make_gather(B * N, D, B * npoint, chunk)(vals_rows, idx3)
    new_values = jnp.transpose(gathered.reshape(B, npoint, D), (0, 2, 1))
    return new_coords, new_values


# speedup vs baseline: 50.0105x; 1.0150x over previous
import functools

import jax
import jax.numpy as jnp
from jax import lax
from jax.experimental import pallas as pl
from jax.experimental.pallas import tpu as pltpu
from jax.experimental.pallas import tpu_sc as plsc


_FLUSH = 128
_TAIL = 128


_W = 128
_GROUPS = 4


def _fold(a, b):
    keep = a[0] >= b[0]
    return tuple(jnp.where(keep, ai, bi) for ai, bi in zip(a, b))


def _fold_tb(a, b):
    keep = (a[0] > b[0]) | ((a[0] == b[0]) & (a[1] < b[1]))
    return tuple(jnp.where(keep, ai, bi) for ai, bi in zip(a, b))


_SB = 16
_UNROLL = 4


def _fps_body(npoint, x_ref, y_ref, z_ref, idx_ref, cx_ref, cy_ref, cz_ref, dist_ref):
    B, N = x_ref.shape
    ncols = N // _W
    cpg = ncols // _GROUPS
    sb = _SB if B >= _SB else B
    nsets = B // sb

    dist_ref[...] = jnp.full((B, N), 1e10, dtype=jnp.float32)

    lane_f = lax.broadcasted_iota(jnp.int32, (sb, _FLUSH), 1)
    consts = []
    st0 = []
    for s in range(nsets):
        rows = slice(s * sb, (s + 1) * sb)
        boffs_f = (
            lax.broadcasted_iota(jnp.int32, (sb, 1), 0) + (s * sb)
        ).astype(jnp.float32) * float(N)
        lane_wf = (
            lax.broadcasted_iota(jnp.int32, (sb, _W), 1).astype(jnp.float32) + boffs_f
        )
        consts.append(lane_wf)
        bf0 = jnp.zeros((sb, _FLUSH), dtype=jnp.float32)
        st0.append(
            (
                jnp.zeros((sb, 1), dtype=jnp.float32) + boffs_f,
                x_ref[rows, 0:1],
                y_ref[rows, 0:1],
                z_ref[rows, 0:1],
            )
        )

    bf0 = jnp.zeros((sb, _FLUSH), dtype=jnp.float32)

    def step(s, j, st):
        rows = slice(s * sb, (s + 1) * sb)
        far, cx, cy, cz, bi, bx, by, bz = st
        sel = lane_f == j
        bi = jnp.where(sel, far, bi)
        bx = jnp.where(sel, cx, bx)
        by = jnp.where(sel, cy, by)
        bz = jnp.where(sel, cz, bz)
        bests = []
        for g in range(_GROUPS):
            best = None
            for c in range(g * cpg, (g + 1) * cpg):
                sl = slice(c * _W, (c + 1) * _W)
                xc = x_ref[rows, sl]
                yc = y_ref[rows, sl]
                zc = z_ref[rows, sl]
                dx = xc - cx
                dy = yc - cy
                dz = zc - cz
                d = (dx * dx + dz * dz) + dy * dy
                dc = jnp.minimum(dist_ref[rows, sl], d)
                dist_ref[rows, sl] = dc
                col = (dc, jnp.full((sb, _W), float(c), jnp.float32), xc, yc, zc)
                best = col if best is None else _fold(best, col)
            bests.append(best)
        best = _fold(_fold(bests[0], bests[1]), _fold(bests[2], bests[3]))
        v, ci, px, py, pz = best
        gi = ci * float(_W) + consts[s]
        m = jnp.max(v, axis=1, keepdims=True)
        eq = v == m
        zero = jnp.zeros_like(v)
        one = jnp.ones_like(v)
        nf = jnp.min(jnp.where(eq, gi, jnp.float32(2.0**30)), axis=1, keepdims=True)
        sx = jnp.sum(jnp.where(eq, px, zero), axis=1, keepdims=True)
        sy = jnp.sum(jnp.where(eq, py, zero), axis=1, keepdims=True)
        sz = jnp.sum(jnp.where(eq, pz, zero), axis=1, keepdims=True)
        cnt = jnp.sum(jnp.where(eq, one, zero), axis=1, keepdims=True)

        def _fix(args):
            nf_, gi_, px_, py_, pz_, _sx, _sy, _sz = args
            oh = gi_ == nf_
            z = jnp.zeros_like(px_)
            return (
                jnp.sum(jnp.where(oh, px_, z), axis=1, keepdims=True),
                jnp.sum(jnp.where(oh, py_, z), axis=1, keepdims=True),
                jnp.sum(jnp.where(oh, pz_, z), axis=1, keepdims=True),
            )

        def _keep(args):
            return (args[5], args[6], args[7])

        tie = jnp.any(cnt > 1.5)
        ncx, ncy, ncz = lax.cond(tie, _fix, _keep, (nf, gi, px, py, pz, sx, sy, sz))
        return (nf, ncx, ncy, ncz, bi, bx, by, bz)

    def body2(jj, carry):
        for u in range(_UNROLL):
            j = jj * _UNROLL + u
            carry = tuple(step(s, j, carry[s]) for s in range(nsets))
        return carry

    carry = tuple(st0[s] + (bf0, bf0, bf0, bf0) for s in range(nsets))
    for blk in range(npoint // _FLUSH):
        carry = lax.fori_loop(0, _FLUSH // _UNROLL, body2, carry)
        sl = slice(blk * _FLUSH, (blk + 1) * _FLUSH)
        new_carry = []
        for s in range(nsets):
            rows = slice(s * sb, (s + 1) * sb)
            far, cx, cy, cz, bi, bx, by, bz = carry[s]
            idx_ref[rows, sl] = bi.astype(jnp.int32)
            cx_ref[rows, sl] = bx
            cy_ref[rows, sl] = by
            cz_ref[rows, sl] = bz
            new_carry.append((far, cx, cy, cz, bf0, bf0, bf0, bf0))
        carry = tuple(new_carry)


def _run_fps(x, y, z, npoint):
    B, N = x.shape
    out_shape = [jax.ShapeDtypeStruct((B, npoint), jnp.int32)] + [
        jax.ShapeDtypeStruct((B, npoint), jnp.float32)
    ] * 3
    return pl.pallas_call(
        functools.partial(_fps_body, npoint),
        out_shape=out_shape,
        scratch_shapes=[pltpu.VMEM((B, N), jnp.float32)],
    )(x, y, z)


def _make_gather(n_rows, D, n_idx, chunk=128):
    NW = 32
    per_w = n_idx // (NW * chunk)
    mesh = plsc.VectorSubcoreMesh(core_axis_name="c", subcore_axis_name="s")

    @functools.partial(
        pl.kernel,
        mesh=mesh,
        compiler_params=pltpu.CompilerParams(use_tc_tiling_on_sc=False),
        out_type=jax.ShapeDtypeStruct((NW, per_w, chunk, D), jnp.float32),
        scratch_types=[
            pltpu.VMEM((per_w, chunk), jnp.int32),
            pltpu.VMEM((per_w, chunk, D), jnp.float32),
            pltpu.SemaphoreType.DMA,
        ],
    )
    def g(vals_hbm, idx_hbm, out_hbm, idx_v, rows_v, sem):
        wid = lax.axis_index("s") * 2 + lax.axis_index("c")
        pltpu.sync_copy(idx_hbm.at[wid], idx_v)
        cps = [
            pltpu.async_copy(vals_hbm.at[idx_v.at[j]], rows_v.at[j], sem)
            for j in range(per_w)
        ]
        for cp in cps:
            cp.wait()
        pltpu.sync_copy(rows_v, out_hbm.at[wid])

    return g


def kernel(coords, values):
    B, C, N = coords.shape
    _, D, _ = values.shape
    npoint = N // 2

    x = coords[:, 0, :]
    y = coords[:, 1, :]
    z = coords[:, 2, :]
    idx_g, cx, cy, cz = _run_fps(x, y, z, npoint)
    new_coords = jnp.stack([cx, cy, cz], axis=1)

    vals_rows = jnp.transpose(values, (0, 2, 1)).reshape(B * N, D)
    chunk = 128
    idx3 = idx_g.reshape(32, (B * npoint) // (32 * chunk), chunk)
    gathered = _make_gather(B * N, D, B * npoint, chunk)(vals_rows, idx3)
    new_values = jnp.transpose(gathered.reshape(B, npoint, D), (0, 2, 1))
    return new_coords, new_values


# speedup vs baseline: 50.2565x; 1.0049x over previous
import functools

import jax
import jax.numpy as jnp
from jax import lax
from jax.experimental import pallas as pl
from jax.experimental.pallas import tpu as pltpu
from jax.experimental.pallas import tpu_sc as plsc


_FLUSH = 128
_TAIL = 128


_W = 128
_GROUPS = 4


def _fold(a, b):
    keep = a[0] >= b[0]
    return tuple(jnp.where(keep, ai, bi) for ai, bi in zip(a, b))


def _fold_tb(a, b):
    keep = (a[0] > b[0]) | ((a[0] == b[0]) & (a[1] < b[1]))
    return tuple(jnp.where(keep, ai, bi) for ai, bi in zip(a, b))


_SB = 16
_UNROLL = 4


def _fps_body(npoint, x_ref, y_ref, z_ref, idx_ref, cx_ref, cy_ref, cz_ref, *dist_refs):
    B, N = x_ref.shape
    ncols = N // _W
    cpg = ncols // _GROUPS
    sb = _SB if B >= _SB else B
    nsets = B // sb

    for dr in dist_refs:
        dr[...] = jnp.full((sb, N), 1e10, dtype=jnp.float32)

    lane_f = lax.broadcasted_iota(jnp.int32, (sb, _FLUSH), 1)
    consts = []
    for s in range(nsets):
        boffs_f = (
            lax.broadcasted_iota(jnp.int32, (sb, 1), 0) + (s * sb)
        ).astype(jnp.float32) * float(N)
        lane_wf = (
            lax.broadcasted_iota(jnp.int32, (sb, _W), 1).astype(jnp.float32) + boffs_f
        )
        consts.append((boffs_f, lane_wf))

    bf0 = jnp.zeros((sb, _FLUSH), dtype=jnp.float32)

    def dense(s, c3):
        rows = slice(s * sb, (s + 1) * sb)
        cx, cy, cz = c3
        bests = []
        for g in range(_GROUPS):
            best = None
            for c in range(g * cpg, (g + 1) * cpg):
                sl = slice(c * _W, (c + 1) * _W)
                xc = x_ref[rows, sl]
                yc = y_ref[rows, sl]
                zc = z_ref[rows, sl]
                dx = xc - cx
                dy = yc - cy
                dz = zc - cz
                d = (dx * dx + dz * dz) + dy * dy
                dc = jnp.minimum(dist_refs[s][:, sl], d)
                dist_refs[s][:, sl] = dc
                col = (dc, jnp.full((sb, _W), float(c), jnp.float32), xc, yc, zc)
                best = col if best is None else _fold(best, col)
            bests.append(best)
        return _fold(_fold(bests[0], bests[1]), _fold(bests[2], bests[3]))

    def tail_pre(s, best):
        v, ci, px, py, pz = best
        gi = ci * float(_W) + consts[s][1]
        m = jnp.max(v, axis=1, keepdims=True)
        eq = v == m
        zero = jnp.zeros_like(v)
        one = jnp.ones_like(v)
        nf = jnp.min(jnp.where(eq, gi, jnp.float32(2.0**30)), axis=1, keepdims=True)
        sx = jnp.sum(jnp.where(eq, px, zero), axis=1, keepdims=True)
        sy = jnp.sum(jnp.where(eq, py, zero), axis=1, keepdims=True)
        sz = jnp.sum(jnp.where(eq, pz, zero), axis=1, keepdims=True)
        cnt = jnp.sum(jnp.where(eq, one, zero), axis=1, keepdims=True)
        return (nf, gi, px, py, pz, sx, sy, sz, cnt)

    def tail_fix(pre):
        nf, gi, px, py, pz, sx, sy, sz, cnt = pre

        def _fix(args):
            nf_, gi_, px_, py_, pz_, _sx, _sy, _sz = args
            oh = gi_ == nf_
            z = jnp.zeros_like(px_)
            return (
                jnp.sum(jnp.where(oh, px_, z), axis=1, keepdims=True),
                jnp.sum(jnp.where(oh, py_, z), axis=1, keepdims=True),
                jnp.sum(jnp.where(oh, pz_, z), axis=1, keepdims=True),
            )

        def _keep(args):
            return (args[5], args[6], args[7])

        tie = jnp.any(cnt > 1.5)
        ncx, ncy, ncz = lax.cond(tie, _fix, _keep, (nf, gi, px, py, pz, sx, sy, sz))
        return nf, (ncx, ncy, ncz)

    def tail(s, best):
        return tail_fix(tail_pre(s, best))

    def bufput(bufs, j, far, c3):
        sel = lane_f == j
        return (
            jnp.where(sel, far, bufs[0]),
            jnp.where(sel, c3[0], bufs[1]),
            jnp.where(sel, c3[1], bufs[2]),
            jnp.where(sel, c3[2], bufs[3]),
        )

    far_i = []
    c_i = []
    for s in range(nsets):
        rows = slice(s * sb, (s + 1) * sb)
        far_i.append(jnp.zeros((sb, 1), dtype=jnp.float32) + consts[s][0])
        c_i.append((x_ref[rows, 0:1], y_ref[rows, 0:1], z_ref[rows, 0:1]))

    if nsets == 1:
        carry = (dense(0, c_i[0]), far_i[0], c_i[0], (bf0, bf0, bf0, bf0))

        def bodyU(t, carry):
            best0, far0, c0, bufs0 = carry
            for u in range(_UNROLL):
                j = t * _UNROLL + u
                bufs0 = bufput(bufs0, j % _FLUSH, far0, c0)
                pre = tail_pre(0, best0)
                far0, c0 = tail_fix(pre)
                best0 = dense(0, c0)
            return (best0, far0, c0, bufs0)

        spb = _FLUSH // _UNROLL
        for blk in range(npoint // _FLUSH):
            carry = lax.fori_loop(blk * spb, (blk + 1) * spb, bodyU, carry)
            best0, far0, c0, bufs0 = carry
            sl = slice(blk * _FLUSH, (blk + 1) * _FLUSH)
            idx_ref[:, sl] = bufs0[0].astype(jnp.int32)
            cx_ref[:, sl] = bufs0[1]
            cy_ref[:, sl] = bufs0[2]
            cz_ref[:, sl] = bufs0[3]
            carry = (best0, far0, c0, (bf0, bf0, bf0, bf0))
        return

    best0 = dense(0, c_i[0])
    bufs0 = bufput((bf0, bf0, bf0, bf0), 0, far_i[0], c_i[0])
    bufs1 = bufput((bf0, bf0, bf0, bf0), 0, far_i[1], c_i[1])
    far1, c1 = far_i[1], c_i[1]

    def body(j, carry):
        best0, far1, c1, bufs0, bufs1, far0c, c0c = carry
        pre0 = tail_pre(0, best0)
        best1 = dense(1, c1)
        far0, c0 = tail_fix(pre0)
        pre1 = tail_pre(1, best1)
        nbest0 = dense(0, c0)
        nfar1, nc1 = tail_fix(pre1)
        bufs0 = bufput(bufs0, j + 1, far0, c0)
        bufs1 = bufput(bufs1, j + 1, nfar1, nc1)
        return (nbest0, nfar1, nc1, bufs0, bufs1, far0, c0)

    carry = (best0, far1, c1, bufs0, bufs1, far_i[0], c_i[0])
    for blk in range(npoint // _FLUSH):
        carry = lax.fori_loop(0, _FLUSH, body,
                              carry if blk == 0 else carry)
        best0, far1, c1, bufs0, bufs1, far0c, c0c = carry
        sl = slice(blk * _FLUSH, (blk + 1) * _FLUSH)
        idx_ref[0:sb, sl] = bufs0[0].astype(jnp.int32)
        cx_ref[0:sb, sl] = bufs0[1]
        cy_ref[0:sb, sl] = bufs0[2]
        cz_ref[0:sb, sl] = bufs0[3]
        idx_ref[sb : 2 * sb, sl] = bufs1[0].astype(jnp.int32)
        cx_ref[sb : 2 * sb, sl] = bufs1[1]
        cy_ref[sb : 2 * sb, sl] = bufs1[2]
        cz_ref[sb : 2 * sb, sl] = bufs1[3]
        bufs0 = bufput((bf0, bf0, bf0, bf0), 0, far0c, c0c)
        bufs1 = bufput((bf0, bf0, bf0, bf0), 0, far1, c1)
        carry = (best0, far1, c1, bufs0, bufs1, far0c, c0c)


def _run_fps(x, y, z, npoint):
    B, N = x.shape
    out_shape = [jax.ShapeDtypeStruct((B, npoint), jnp.int32)] + [
        jax.ShapeDtypeStruct((B, npoint), jnp.float32)
    ] * 3
    sb = _SB if B >= _SB else B
    return pl.pallas_call(
        functools.partial(_fps_body, npoint),
        out_shape=out_shape,
        scratch_shapes=[pltpu.VMEM((sb, N), jnp.float32) for _ in range(B // sb)],
    )(x, y, z)


def _make_gather(n_rows, D, n_idx, chunk=128):
    NW = 32
    per_w = n_idx // (NW * chunk)
    mesh = plsc.VectorSubcoreMesh(core_axis_name="c", subcore_axis_name="s")

    @functools.partial(
        pl.kernel,
        mesh=mesh,
        compiler_params=pltpu.CompilerParams(use_tc_tiling_on_sc=False),
        out_type=jax.ShapeDtypeStruct((NW, per_w, chunk, D), jnp.float32),
        scratch_types=[
            pltpu.VMEM((per_w, chunk), jnp.int32),
            pltpu.VMEM((per_w, chunk, D), jnp.float32),
            pltpu.SemaphoreType.DMA,
        ],
    )
    def g(vals_hbm, idx_hbm, out_hbm, idx_v, rows_v, sem):
        wid = lax.axis_index("s") * 2 + lax.axis_index("c")
        pltpu.sync_copy(idx_hbm.at[wid], idx_v)
        cps = [
            pltpu.async_copy(vals_hbm.at[idx_v.at[j]], rows_v.at[j], sem)
            for j in range(per_w)
        ]
        for cp in cps:
            cp.wait()
        pltpu.sync_copy(rows_v, out_hbm.at[wid])

    return g


def kernel(coords, values):
    B, C, N = coords.shape
    _, D, _ = values.shape
    npoint = N // 2

    x = coords[:, 0, :]
    y = coords[:, 1, :]
    z = coords[:, 2, :]
    idx_g, cx, cy, cz = _run_fps(x, y, z, npoint)
    new_coords = jnp.stack([cx, cy, cz], axis=1)

    vals_rows = jnp.transpose(values, (0, 2, 1)).reshape(B * N, D)
    chunk = 128
    idx3 = idx_g.reshape(32, (B * npoint) // (32 * chunk), chunk)
    gathered = _make_gather(B * N, D, B * npoint, chunk)(vals_rows, idx3)
    new_values = jnp.transpose(gathered.reshape(B, npoint, D), (0, 2, 1))
    return new_coords, new_values
